# baseline (speedup 1.0000x reference)
import jax
import jax.numpy as jnp
from jax import lax
from jax.experimental import pallas as pl
from jax.experimental.pallas import tpu as pltpu

_TBL_H = 15
_TBL_W = 15
_B = 4
_R = 8


def _stream_kernel(i0_ref, i1_ref, tbl_ref, x_ref, o_ref,
                   bias_ref, in_ref, out_ref, in_sems, out_sems):
    kflat = i0_ref[...] * _TBL_W + i1_ref[...]

    def tbl_body(t, acc):
        v = tbl_ref[t // _TBL_W, t % _TBL_W]
        return acc + jnp.where(kflat == t, v, 0.0)

    bias_ref[...] = lax.fori_loop(
        0, _TBL_H * _TBL_W, tbl_body, jnp.zeros(kflat.shape, jnp.float32)
    )

    nsteps = x_ref.shape[0] // _B

    def in_copy(i, slot):
        return pltpu.make_async_copy(
            x_ref.at[pl.ds(i * _B, _B)], in_ref.at[slot], in_sems.at[slot])

    def out_copy(i, slot):
        return pltpu.make_async_copy(
            out_ref.at[slot], o_ref.at[pl.ds(i * _B, _B)], out_sems.at[slot])

    for r in range(_R):
        in_copy(r, r).start()

    def step(i, carry):
        slot = lax.rem(i, _R)
        in_copy(i, slot).wait()

        @pl.when(i >= _R)
        def _wait_out_slot():
            out_copy(i - _R, slot).wait()

        out_ref[slot] = in_ref[slot] + bias_ref[...][None, None, :, :]
        out_copy(i, slot).start()

        @pl.when(i + _R < nsteps)
        def _prefetch():
            in_copy(i + _R, slot).start()

        return carry

    lax.fori_loop(0, nsteps, step, 0)

    for r in range(_R):
        i = nsteps - _R + r
        out_copy(i, i % _R).wait()


def kernel(x, pos_embeddings, relative_indices):
    nb, nh, h, w = x.shape
    i0 = relative_indices[:, :, 0]
    i1 = relative_indices[:, :, 1]

    out = pl.pallas_call(
        _stream_kernel,
        in_specs=[
            pl.BlockSpec(memory_space=pltpu.VMEM),
            pl.BlockSpec(memory_space=pltpu.VMEM),
            pl.BlockSpec(memory_space=pltpu.SMEM),
            pl.BlockSpec(memory_space=pl.ANY),
        ],
        out_specs=pl.BlockSpec(memory_space=pl.ANY),
        out_shape=jax.ShapeDtypeStruct(x.shape, jnp.float32),
        scratch_shapes=[
            pltpu.VMEM((h, w), jnp.float32),
            pltpu.VMEM((_R, _B, nh, h, w), jnp.float32),
            pltpu.VMEM((_R, _B, nh, h, w), jnp.float32),
            pltpu.SemaphoreType.DMA((_R,)),
            pltpu.SemaphoreType.DMA((_R,)),
        ],
    )(i0, i1, pos_embeddings, x)
    return out
